# Initial kernel scaffold; baseline (speedup 1.0000x reference)
import jax
import jax.numpy as jnp
from jax.experimental import pallas as pl


def kernel(verts, faces):
    raise NotImplementedError("write your pallas kernel here")


# speedup vs baseline: 32.6021x; 32.6021x over previous
import functools

import jax
import jax.numpy as jnp
from jax import lax
from jax.experimental import pallas as pl
from jax.experimental.pallas import tpu as pltpu
from jax.experimental.pallas import tpu_sc as plsc

NC = 2
NS = 16
NW = NC * NS
L = 16
BS = 128


def _sc_body(J, VCH, verts_hbm, faces_hbm, out_hbm,
             ia, ib, ic,
             gax, gay, gaz, gbx, gby, gbz, gcx, gcy, gcz,
             nx, ny, nz, zbuf,
             vtx, vty, vtz, accx, accy, accz, sem):
    c = lax.axis_index("c")
    s = lax.axis_index("s")
    w = c * NS + s
    VACC = VCH * NS

    zv = jnp.zeros((L,), jnp.float32)

    def _zfill(k, _):
        zbuf[pl.ds(k * L, L)] = zv
        return _

    lax.fori_loop(0, VCH // L, _zfill, None)
    base_v = s * VCH
    sl = pl.ds(base_v, VCH)
    pltpu.sync_copy(zbuf, accx.at[sl])
    pltpu.sync_copy(zbuf, accy.at[sl])
    pltpu.sync_copy(zbuf, accz.at[sl])
    pltpu.sync_copy(verts_hbm.at[pl.ds(0 * VACC + base_v, VCH)], vtx.at[sl])
    pltpu.sync_copy(verts_hbm.at[pl.ds(1 * VACC + base_v, VCH)], vty.at[sl])
    pltpu.sync_copy(verts_hbm.at[pl.ds(2 * VACC + base_v, VCH)], vtz.at[sl])

    pltpu.sync_copy(faces_hbm.at[0, w], ia)
    pltpu.sync_copy(faces_hbm.at[1, w], ib)
    pltpu.sync_copy(faces_hbm.at[2, w], ic)

    plsc.subcore_barrier()

    def _step(j, _):
        ra, rb, rc = ia.at[j], ib.at[j], ic.at[j]
        ds = [
            pltpu.async_copy(vtx.at[ra], gax, sem),
            pltpu.async_copy(vty.at[ra], gay, sem),
            pltpu.async_copy(vtz.at[ra], gaz, sem),
            pltpu.async_copy(vtx.at[rb], gbx, sem),
            pltpu.async_copy(vty.at[rb], gby, sem),
            pltpu.async_copy(vtz.at[rb], gbz, sem),
            pltpu.async_copy(vtx.at[rc], gcx, sem),
            pltpu.async_copy(vty.at[rc], gcy, sem),
            pltpu.async_copy(vtz.at[rc], gcz, sem),
        ]
        for d in ds:
            d.wait()
        for i in range(BS // L):
            ii = pl.ds(i * L, L)
            ax, ay, az = gax[ii], gay[ii], gaz[ii]
            bx, by, bz = gbx[ii], gby[ii], gbz[ii]
            cx, cy, cz = gcx[ii], gcy[ii], gcz[ii]
            ux, uy, uz = cx - bx, cy - by, cz - bz
            vx, vy, vz = ax - bx, ay - by, az - bz
            nx[ii] = uy * vz - uz * vy
            ny[ii] = uz * vx - ux * vz
            nz[ii] = ux * vy - uy * vx
        for row in (ra, rb, rc):
            pltpu.sync_copy(nx, accx.at[row], add=True)
            pltpu.sync_copy(ny, accy.at[row], add=True)
            pltpu.sync_copy(nz, accz.at[row], add=True)
        return _

    lax.fori_loop(0, J, _step, None)

    plsc.subcore_barrier()

    pltpu.sync_copy(accx.at[sl], out_hbm.at[pl.ds((c * 3 + 0) * VACC + base_v, VCH)])
    pltpu.sync_copy(accy.at[sl], out_hbm.at[pl.ds((c * 3 + 1) * VACC + base_v, VCH)])
    pltpu.sync_copy(accz.at[sl], out_hbm.at[pl.ds((c * 3 + 2) * VACC + base_v, VCH)])


def _tc_norm(V, q_ref, o_ref):
    p = q_ref[0] + q_ref[1]
    ss = jnp.sum(p * p, axis=0, keepdims=True)
    scale = 1.0 / jnp.maximum(jnp.sqrt(ss), 1e-6)
    o_ref[...] = (p * scale)[:, :V]


def kernel(verts, faces):
    V = verts.shape[0]
    F = faces.shape[0]
    J = -(-F // (NW * BS))
    J = -(-J // 8) * 8
    Fp = NW * J * BS
    VCH = (-(-V // (NS * BS))) * BS
    if VCH * NS == V:
        VCH += BS
    VACC = VCH * NS

    faces = faces.astype(jnp.int32)
    pad_idx = V + (jnp.arange(Fp - F, dtype=jnp.int32) % (VACC - V))
    f_pad = jnp.concatenate(
        [faces, jnp.broadcast_to(pad_idx[:, None], (Fp - F, 3))], axis=0)
    f_soa = f_pad.T.reshape(3, NW, J, BS)
    v_soa = jnp.zeros((3, VACC), jnp.float32).at[:, :V].set(verts.T).reshape(-1)

    mesh = plsc.VectorSubcoreMesh(
        core_axis_name="c", subcore_axis_name="s",
        num_cores=NC, num_subcores=NS)
    sc = pl.kernel(
        functools.partial(_sc_body, J, VCH),
        out_type=jax.ShapeDtypeStruct((NC * 3 * VACC,), jnp.float32),
        mesh=mesh,
        scratch_types=[
            pltpu.VMEM((J, BS), jnp.int32),
            pltpu.VMEM((J, BS), jnp.int32),
            pltpu.VMEM((J, BS), jnp.int32),
        ] + [pltpu.VMEM((BS,), jnp.float32)] * 12
        + [
            pltpu.VMEM((VCH,), jnp.float32),
            pltpu.VMEM_SHARED((VACC,), jnp.float32),
            pltpu.VMEM_SHARED((VACC,), jnp.float32),
            pltpu.VMEM_SHARED((VACC,), jnp.float32),
            pltpu.VMEM_SHARED((VACC,), jnp.float32),
            pltpu.VMEM_SHARED((VACC,), jnp.float32),
            pltpu.VMEM_SHARED((VACC,), jnp.float32),
            pltpu.SemaphoreType.DMA,
        ],
    )
    partials = sc(v_soa, f_soa).reshape(NC, 3, VACC)

    out = pl.pallas_call(
        functools.partial(_tc_norm, V),
        out_shape=jax.ShapeDtypeStruct((3, V), jnp.float32),
    )(partials)
    return out.T
